# baseline (speedup 1.0000x reference)
import functools

import jax
import jax.numpy as jnp
from jax import lax
from jax.experimental import pallas as pl
from jax.experimental.pallas import tpu as pltpu
from jax.experimental.pallas import tpu_sc as plsc

N = 10000
D = 128
BM = 400


def _sc_gather(table, idx_pad, bpad, b_per_w, nc):
    mesh = plsc.VectorSubcoreMesh(core_axis_name="c", subcore_axis_name="s")

    @functools.partial(
        pl.kernel,
        mesh=mesh,
        out_type=jax.ShapeDtypeStruct((bpad, D), jnp.float32),
        scratch_types=[
            pltpu.VMEM((b_per_w,), jnp.int32),
            pltpu.VMEM((b_per_w, D), jnp.float32),
            pltpu.SemaphoreType.DMA,
        ],
    )
    def gk(table_hbm, idx_hbm, out_hbm, idx_v, rows_v, sem):
        wid = lax.axis_index("s") * nc + lax.axis_index("c")
        base = wid * b_per_w
        pltpu.sync_copy(idx_hbm.at[pl.ds(base, b_per_w)], idx_v)
        pltpu.async_copy(table_hbm.at[idx_v], rows_v, sem).wait()
        pltpu.sync_copy(rows_v, out_hbm.at[pl.ds(base, b_per_w)])

    return gk(table, idx_pad)


def _prep_body(e_ref, w1_ref, s1_ref):
    s1 = jnp.dot(e_ref[...], w1_ref[...], preferred_element_type=jnp.float32)
    s1_ref[...] = s1.astype(jnp.bfloat16)


def _prep(e, w1):
    return pl.pallas_call(
        _prep_body,
        out_shape=jax.ShapeDtypeStruct((N, D), jnp.bfloat16),
    )(e, w1)


def _v_body(adj_ref, s1_ref, b1_ref, w2_ref, lw1_ref, lw2r_ref, v_ref):
    a = adj_ref[...].astype(jnp.bfloat16)
    acc = jnp.dot(a, s1_ref[...], preferred_element_type=jnp.float32)
    h = jnp.maximum(acc + b1_ref[...], 0.0)
    u = jnp.dot(h, w2_ref[...], preferred_element_type=jnp.float32)
    u = jnp.dot(u, lw1_ref[...], preferred_element_type=jnp.float32)
    v_ref[...] = jnp.sum(u * lw2r_ref[...], axis=1, keepdims=True)


def _v_pass(adj, s1b, b1row, w2, lw1, lw2r):
    return pl.pallas_call(
        _v_body,
        grid=(N // BM,),
        in_specs=[
            pl.BlockSpec((BM, N), lambda i: (i, 0)),
            pl.BlockSpec((N, D), lambda i: (0, 0)),
            pl.BlockSpec((1, D), lambda i: (0, 0)),
            pl.BlockSpec((D, D), lambda i: (0, 0)),
            pl.BlockSpec((D, 16), lambda i: (0, 0)),
            pl.BlockSpec((1, 16), lambda i: (0, 0)),
        ],
        out_specs=pl.BlockSpec((BM, 1), lambda i: (i, 0)),
        out_shape=jax.ShapeDtypeStruct((N, 1), jnp.float32),
    )(adj, s1b, b1row, w2, lw1, lw2r)


def _x_body(adj_ref, vt_ref, b2_ref, lb1_ref, lw1_ref, lw2r_ref, lb2_ref, x_ref):
    a = adj_ref[...]
    xs = jnp.sum(a * vt_ref[...], axis=1, keepdims=True)
    t = jnp.dot(b2_ref[...], lw1_ref[...], preferred_element_type=jnp.float32)
    t = t + lb1_ref[...]
    c = jnp.sum(t * lw2r_ref[...], axis=1, keepdims=True) + lb2_ref[...]
    x_ref[...] = xs + c


def _x_pass(adj, vt, b2row, lb1row, lw1, lw2r, lb2row):
    return pl.pallas_call(
        _x_body,
        grid=(N // BM,),
        in_specs=[
            pl.BlockSpec((BM, N), lambda i: (i, 0)),
            pl.BlockSpec((1, N), lambda i: (0, 0)),
            pl.BlockSpec((1, D), lambda i: (0, 0)),
            pl.BlockSpec((1, 16), lambda i: (0, 0)),
            pl.BlockSpec((D, 16), lambda i: (0, 0)),
            pl.BlockSpec((1, 16), lambda i: (0, 0)),
            pl.BlockSpec((1, 1), lambda i: (0, 0)),
        ],
        out_specs=pl.BlockSpec((BM, 1), lambda i: (i, 0)),
        out_shape=jax.ShapeDtypeStruct((N, 1), jnp.float32),
    )(adj, vt, b2row, lb1row, lw1, lw2r, lb2row)


def kernel(features, adj, emb_table, W1, b1, W2, b2, lw1, lb1, lw2, lb2):
    info = plsc.get_sparse_core_info()
    nw = info.num_cores * info.num_subcores
    bpad = ((N + 8 * nw - 1) // (8 * nw)) * (8 * nw)
    b_per_w = bpad // nw

    idx = features.astype(jnp.int32)
    idx_pad = jnp.pad(idx, (0, bpad - N))
    e_pad = _sc_gather(emb_table, idx_pad, bpad, b_per_w, info.num_cores)
    e = e_pad[:N]

    s1b = _prep(e, W1)
    v = _v_pass(adj, s1b, b1.reshape(1, D), W2, lw1, lw2.reshape(1, 16))
    x = _x_pass(adj, v.reshape(1, N), b2.reshape(1, D), lb1.reshape(1, 16),
                lw1, lw2.reshape(1, 16), lb2.reshape(1, 1))
    return (x, e)
